# baseline (speedup 1.0000x reference)
import functools

import jax
import jax.numpy as jnp
from jax import lax
from jax.experimental import pallas as pl
from jax.experimental.pallas import tpu as pltpu
from jax.experimental.pallas import tpu_sc as plsc

N = 10000
E = 320000
D = 128
G = 64
EPS = 1e-5

NC = 2
NS = 16
NW = NC * NS
N_PAD = 10240
RPT = N_PAD // NS
EPW = 10368
E_PAD = EPW * NW
CH = 32
NCHUNK = EPW // CH
NSTAGE = 4
CPS = NCHUNK // NSTAGE
NPAIR = 3
KITER = CPS // NPAIR

def _edge_acc_body(yin_hbm, yout_hbm, src_hbm, dst_hbm, zero_hbm, out_hbm,
                   sd_v, rows, gsems, ssems, acc_sh):
    cid = lax.axis_index("c")
    sid = lax.axis_index("s")
    wid = sid * NC + cid

    row0 = sid * RPT
    pltpu.sync_copy(zero_hbm.at[pl.ds(row0, RPT)], acc_sh.at[pl.ds(row0, RPT)])

    plsc.subcore_barrier()

    def gathers(c, p):
        pltpu.async_copy(yin_hbm.at[sd_v.at[0, c]], rows.at[2 * p],
                         gsems.at[2 * p])
        pltpu.async_copy(yout_hbm.at[sd_v.at[1, c]], rows.at[2 * p + 1],
                         gsems.at[2 * p + 1])

    def wait_scatters(c, p):
        pltpu.make_async_copy(rows.at[2 * p], acc_sh.at[sd_v.at[1, c]],
                              ssems.at[2 * p]).wait()
        pltpu.make_async_copy(rows.at[2 * p + 1], acc_sh.at[sd_v.at[0, c]],
                              ssems.at[2 * p + 1]).wait()

    def stage(st, carry):
        pltpu.sync_copy(src_hbm.at[wid, st], sd_v.at[0])
        pltpu.sync_copy(dst_hbm.at[wid, st], sd_v.at[1])
        for p in range(NPAIR):
            gathers(p, p)

        def kbody(k, c):
            for j in range(NPAIR):
                ci = NPAIR * k + j
                sa = sd_v.at[0, ci]
                da = sd_v.at[1, ci]
                pltpu.make_async_copy(yin_hbm.at[sa], rows.at[2 * j],
                                      gsems.at[2 * j]).wait()
                pltpu.async_copy(rows.at[2 * j], acc_sh.at[da],
                                 ssems.at[2 * j], add=True)
                pltpu.make_async_copy(yout_hbm.at[da], rows.at[2 * j + 1],
                                      gsems.at[2 * j + 1]).wait()
                pltpu.async_copy(rows.at[2 * j + 1], acc_sh.at[sa],
                                 ssems.at[2 * j + 1], add=True)

                cp = ci - (NPAIR - 1)
                pp = (j + 1) % NPAIR

                @pl.when(jnp.logical_and(cp >= 0, cp < CPS - NPAIR))
                def _():
                    wait_scatters(cp, pp)
                    gathers(cp + NPAIR, pp)

            return c

        lax.fori_loop(0, KITER, kbody, 0)

        for p in range(NPAIR):
            wait_scatters(CPS - NPAIR + p, p)
        return carry

    lax.fori_loop(0, NSTAGE, stage, 0)
    plsc.subcore_barrier()

    pltpu.sync_copy(acc_sh.at[pl.ds(row0, RPT)],
                    out_hbm.at[cid, pl.ds(row0, RPT)])


@functools.cache
def _edge_acc():
    mesh = plsc.VectorSubcoreMesh(core_axis_name="c", subcore_axis_name="s")
    return pl.kernel(
        _edge_acc_body,
        mesh=mesh,
        out_type=jax.ShapeDtypeStruct((NC, N_PAD, D), jnp.float32),
        scratch_types=[
            pltpu.VMEM((2, CPS, CH), jnp.int32),
            pltpu.VMEM((2 * NPAIR, CH, D), jnp.float32),
            pltpu.SemaphoreType.DMA((2 * NPAIR,)),
            pltpu.SemaphoreType.DMA((2 * NPAIR,)),
            pltpu.VMEM_SHARED((N_PAD, D), jnp.float32),
        ],
    )


def _mm3_body(x_ref, ws_ref, wi_ref, wo_ref, s_ref, yi_ref, yo_ref):
    xv = x_ref[...]
    s_ref[...] = jnp.dot(xv, ws_ref[...], preferred_element_type=jnp.float32, precision=lax.Precision.HIGHEST)
    yi_ref[pl.ds(0, N), :] = jnp.dot(xv, wi_ref[...], preferred_element_type=jnp.float32, precision=lax.Precision.HIGHEST)
    yo_ref[pl.ds(0, N), :] = jnp.dot(xv, wo_ref[...], preferred_element_type=jnp.float32, precision=lax.Precision.HIGHEST)


_mm3 = pl.pallas_call(
    _mm3_body,
    out_shape=[jax.ShapeDtypeStruct((N, D), jnp.float32),
               jax.ShapeDtypeStruct((N_PAD, D), jnp.float32),
               jax.ShapeDtypeStruct((N_PAD, D), jnp.float32)],
)


def _normalize(s_ref, acc_ref, g_ref, b_ref):
    acc = acc_ref[...]
    h = jnp.maximum(s_ref[...] + acc[0, :N] + acc[1, :N], 0.0)
    mu = jnp.mean(h, axis=0, keepdims=True)
    var = jnp.mean((h - mu) * (h - mu), axis=0, keepdims=True)
    return (h - mu) * lax.rsqrt(var + EPS) * g_ref[...] + b_ref[...]


def _norm_body(s_ref, acc_ref, g_ref, b_ref, h_ref):
    h_ref[...] = _normalize(s_ref, acc_ref, g_ref, b_ref)


_norm = pl.pallas_call(
    _norm_body,
    out_shape=jax.ShapeDtypeStruct((N, D), jnp.float32),
)


def _pool_body(s_ref, acc_ref, g_ref, b_ref, batch_ref, out_ref):
    h = _normalize(s_ref, acc_ref, g_ref, b_ref)
    seg = batch_ref[...]
    gids = lax.broadcasted_iota(jnp.int32, (G, N), 0)
    onehot = (gids == seg).astype(jnp.float32)
    sums = jnp.dot(onehot, h, preferred_element_type=jnp.float32, precision=lax.Precision.HIGHEST)
    counts = jnp.sum(onehot, axis=1, keepdims=True)
    out_ref[...] = sums / jnp.maximum(counts, 1.0)


_pool = pl.pallas_call(
    _pool_body,
    out_shape=jax.ShapeDtypeStruct((G, D), jnp.float32),
)


_pool = pl.pallas_call(
    _pool_body,
    out_shape=jax.ShapeDtypeStruct((G, D), jnp.float32),
)


def kernel(x, edge_index, batch,
           W_self_0, W_in_0, W_out_0, g_0, b_0,
           W_self_1, W_in_1, W_out_1, g_1, b_1,
           W_self_2, W_in_2, W_out_2, g_2, b_2):
    fake = jnp.broadcast_to(
        N + jnp.arange(EPW - E // NW, dtype=jnp.int32) % (N_PAD - N),
        (NW, EPW - E // NW))
    src = jnp.concatenate(
        [edge_index[0].astype(jnp.int32).reshape(NW, E // NW), fake], axis=1)
    dst = jnp.concatenate(
        [edge_index[1].astype(jnp.int32).reshape(NW, E // NW), fake], axis=1)
    src = src.reshape(NW, NSTAGE, CPS, CH)
    dst = dst.reshape(NW, NSTAGE, CPS, CH)
    zero = jnp.zeros((N_PAD, D), jnp.float32)
    batch2 = batch.astype(jnp.int32).reshape(1, N)

    s, yi, yo = _mm3(x, W_self_0, W_in_0, W_out_0)
    acc = _edge_acc()(yi, yo, src, dst, zero)
    h = _norm(s, acc, g_0, b_0)
    s, yi, yo = _mm3(h, W_self_1, W_in_1, W_out_1)
    acc = _edge_acc()(yi, yo, src, dst, zero)
    h = _norm(s, acc, g_1, b_1)
    s, yi, yo = _mm3(h, W_self_2, W_in_2, W_out_2)
    acc = _edge_acc()(yi, yo, src, dst, zero)
    return _pool(s, acc, g_2, b_2, batch2)


# speedup vs baseline: 1.4639x; 1.4639x over previous
import functools

import jax
import jax.numpy as jnp
from jax import lax
from jax.experimental import pallas as pl
from jax.experimental.pallas import tpu as pltpu
from jax.experimental.pallas import tpu_sc as plsc

N = 10000
E = 320000
D = 128
G = 64
EPS = 1e-5

NC = 2
NS = 16
NW = NC * NS
N_PAD = 10240
RPT = N_PAD // NS
EPW = 10240
E_PAD = EPW * NW
CH = 64
NCHUNK = EPW // CH
NSTAGE = 4
CPS = NCHUNK // NSTAGE
NPAIR = 2
KITER = CPS // NPAIR

def _edge_acc_body(yin_hbm, yout_hbm, src_hbm, dst_hbm, zero_hbm, out_hbm,
                   sd_v, rows, gsems, ssems, acc_sh):
    cid = lax.axis_index("c")
    sid = lax.axis_index("s")
    wid = sid * NC + cid

    row0 = sid * RPT
    pltpu.sync_copy(zero_hbm.at[pl.ds(row0, RPT)], acc_sh.at[pl.ds(row0, RPT)])

    plsc.subcore_barrier()

    def gathers(c, p):
        pltpu.async_copy(yin_hbm.at[sd_v.at[0, c]], rows.at[2 * p],
                         gsems.at[2 * p])
        pltpu.async_copy(yout_hbm.at[sd_v.at[1, c]], rows.at[2 * p + 1],
                         gsems.at[2 * p + 1])

    def wait_scatters(c, p):
        pltpu.make_async_copy(rows.at[2 * p], acc_sh.at[sd_v.at[1, c]],
                              ssems.at[2 * p]).wait()
        pltpu.make_async_copy(rows.at[2 * p + 1], acc_sh.at[sd_v.at[0, c]],
                              ssems.at[2 * p + 1]).wait()

    def stage(st, carry):
        pltpu.sync_copy(src_hbm.at[wid, st], sd_v.at[0])
        pltpu.sync_copy(dst_hbm.at[wid, st], sd_v.at[1])
        for p in range(NPAIR):
            gathers(p, p)

        def kbody(k, c):
            for j in range(NPAIR):
                ci = NPAIR * k + j
                sa = sd_v.at[0, ci]
                da = sd_v.at[1, ci]
                pltpu.make_async_copy(yin_hbm.at[sa], rows.at[2 * j],
                                      gsems.at[2 * j]).wait()
                pltpu.async_copy(rows.at[2 * j], acc_sh.at[da],
                                 ssems.at[2 * j], add=True)
                pltpu.make_async_copy(yout_hbm.at[da], rows.at[2 * j + 1],
                                      gsems.at[2 * j + 1]).wait()
                pltpu.async_copy(rows.at[2 * j + 1], acc_sh.at[sa],
                                 ssems.at[2 * j + 1], add=True)

                @pl.when(ci < CPS - NPAIR)
                def _():
                    cn = ci + NPAIR
                    pltpu.make_async_copy(rows.at[2 * j], acc_sh.at[da],
                                          ssems.at[2 * j]).wait()
                    pltpu.async_copy(yin_hbm.at[sd_v.at[0, cn]],
                                     rows.at[2 * j], gsems.at[2 * j])
                    pltpu.make_async_copy(rows.at[2 * j + 1], acc_sh.at[sa],
                                          ssems.at[2 * j + 1]).wait()
                    pltpu.async_copy(yout_hbm.at[sd_v.at[1, cn]],
                                     rows.at[2 * j + 1], gsems.at[2 * j + 1])

            return c

        lax.fori_loop(0, KITER, kbody, 0)

        for p in range(NPAIR):
            wait_scatters(CPS - NPAIR + p, p)
        return carry

    lax.fori_loop(0, NSTAGE, stage, 0)
    plsc.subcore_barrier()

    pltpu.sync_copy(acc_sh.at[pl.ds(row0, RPT)],
                    out_hbm.at[cid, pl.ds(row0, RPT)])


@functools.cache
def _edge_acc():
    mesh = plsc.VectorSubcoreMesh(core_axis_name="c", subcore_axis_name="s")
    return pl.kernel(
        _edge_acc_body,
        mesh=mesh,
        out_type=jax.ShapeDtypeStruct((NC, N_PAD, D), jnp.float32),
        scratch_types=[
            pltpu.VMEM((2, CPS, CH), jnp.int32),
            pltpu.VMEM((2 * NPAIR, CH, D), jnp.float32),
            pltpu.SemaphoreType.DMA((2 * NPAIR,)),
            pltpu.SemaphoreType.DMA((2 * NPAIR,)),
            pltpu.VMEM_SHARED((N_PAD, D), jnp.float32),
        ],
    )


def _mm3_body(x_ref, ws_ref, wi_ref, wo_ref, s_ref, yi_ref, yo_ref):
    xv = x_ref[...]
    s_ref[...] = jnp.dot(xv, ws_ref[...], preferred_element_type=jnp.float32, precision=lax.Precision.HIGHEST)
    yi_ref[pl.ds(0, N), :] = jnp.dot(xv, wi_ref[...], preferred_element_type=jnp.float32, precision=lax.Precision.HIGHEST)
    yo_ref[pl.ds(0, N), :] = jnp.dot(xv, wo_ref[...], preferred_element_type=jnp.float32, precision=lax.Precision.HIGHEST)


_mm3 = pl.pallas_call(
    _mm3_body,
    out_shape=[jax.ShapeDtypeStruct((N, D), jnp.float32),
               jax.ShapeDtypeStruct((N_PAD, D), jnp.float32),
               jax.ShapeDtypeStruct((N_PAD, D), jnp.float32)],
)


def _normalize(s_ref, acc_ref, g_ref, b_ref):
    acc = acc_ref[...]
    h = jnp.maximum(s_ref[...] + acc[0, :N] + acc[1, :N], 0.0)
    mu = jnp.mean(h, axis=0, keepdims=True)
    var = jnp.mean((h - mu) * (h - mu), axis=0, keepdims=True)
    return (h - mu) * lax.rsqrt(var + EPS) * g_ref[...] + b_ref[...]


def _norm_body(s_ref, acc_ref, g_ref, b_ref, h_ref):
    h_ref[...] = _normalize(s_ref, acc_ref, g_ref, b_ref)


_norm = pl.pallas_call(
    _norm_body,
    out_shape=jax.ShapeDtypeStruct((N, D), jnp.float32),
)


def _pool_body(s_ref, acc_ref, g_ref, b_ref, batch_ref, out_ref):
    h = _normalize(s_ref, acc_ref, g_ref, b_ref)
    seg = batch_ref[...]
    gids = lax.broadcasted_iota(jnp.int32, (G, N), 0)
    onehot = (gids == seg).astype(jnp.float32)
    sums = jnp.dot(onehot, h, preferred_element_type=jnp.float32, precision=lax.Precision.HIGHEST)
    counts = jnp.sum(onehot, axis=1, keepdims=True)
    out_ref[...] = sums / jnp.maximum(counts, 1.0)


_pool = pl.pallas_call(
    _pool_body,
    out_shape=jax.ShapeDtypeStruct((G, D), jnp.float32),
)


_pool = pl.pallas_call(
    _pool_body,
    out_shape=jax.ShapeDtypeStruct((G, D), jnp.float32),
)


def kernel(x, edge_index, batch,
           W_self_0, W_in_0, W_out_0, g_0, b_0,
           W_self_1, W_in_1, W_out_1, g_1, b_1,
           W_self_2, W_in_2, W_out_2, g_2, b_2):
    fake = jnp.broadcast_to(
        N + jnp.arange(EPW - E // NW, dtype=jnp.int32) % (N_PAD - N),
        (NW, EPW - E // NW))
    src = jnp.concatenate(
        [edge_index[0].astype(jnp.int32).reshape(NW, E // NW), fake], axis=1)
    dst = jnp.concatenate(
        [edge_index[1].astype(jnp.int32).reshape(NW, E // NW), fake], axis=1)
    src = src.reshape(NW, NSTAGE, CPS, CH)
    dst = dst.reshape(NW, NSTAGE, CPS, CH)
    zero = jnp.zeros((N_PAD, D), jnp.float32)
    batch2 = batch.astype(jnp.int32).reshape(1, N)

    s, yi, yo = _mm3(x, W_self_0, W_in_0, W_out_0)
    acc = _edge_acc()(yi, yo, src, dst, zero)
    h = _norm(s, acc, g_0, b_0)
    s, yi, yo = _mm3(h, W_self_1, W_in_1, W_out_1)
    acc = _edge_acc()(yi, yo, src, dst, zero)
    h = _norm(s, acc, g_1, b_1)
    s, yi, yo = _mm3(h, W_self_2, W_in_2, W_out_2)
    acc = _edge_acc()(yi, yo, src, dst, zero)
    return _pool(s, acc, g_2, b_2, batch2)
